# baseline (speedup 1.0000x reference)
import dataclasses
import functools

import jax
import jax.numpy as jnp
from jax import lax
from jax.experimental import pallas as pl
from jax.experimental.pallas import tpu as pltpu
from jax.experimental.pallas import tpu_sc as plsc

_SAMPLING_EPS = 1e-05
_VT = 2048
_V = 100000
_VP = 100352
_NS = 16
_SH = _VP // _NS
_RADIX = 256
_NEG_INF = float("-inf")


def _proj_keys_body(h_ref, e_ref, key_ref):
    j = pl.program_id(0)
    logits = jax.lax.dot_general(
        h_ref[...], e_ref[...], (((1,), (1,)), ((), ())),
        preferred_element_type=jnp.float32)
    u = jax.lax.bitcast_convert_type(logits + 0.0, jnp.uint32)
    neg = (u >> 31) == 1
    key = jnp.where(neg, u, ~u & jnp.uint32(0x7FFFFFFF))
    col = j * _VT + jax.lax.broadcasted_iota(jnp.int32, key.shape, 1)
    key = jnp.where(col < _V, key, jnp.uint32(0xFFFFFFFF))
    key_ref[...] = jax.lax.bitcast_convert_type(key, jnp.int32)


def _proj_keys(h, emb):
    B, D = h.shape
    grid = (_VP // _VT,)
    return pl.pallas_call(
        _proj_keys_body,
        grid=grid,
        in_specs=[
            pl.BlockSpec((B, D), lambda i: (0, 0)),
            pl.BlockSpec((_VT, D), lambda i: (i, 0)),
        ],
        out_specs=pl.BlockSpec((B, _VT), lambda i: (0, i)),
        out_shape=jax.ShapeDtypeStruct((B, _VP), jnp.int32),
    )(h, emb)


def _key_to_logit(key):
    k = jax.lax.bitcast_convert_type(key, jnp.uint32)
    neg = (k >> 31) == 1
    u = jnp.where(neg, k, ~k & jnp.uint32(0x7FFFFFFF))
    return jax.lax.bitcast_convert_type(u, jnp.float32)


def _sc_sort(keys):
    B = keys.shape[0]
    rows_per_core = B // 2
    cp = pltpu.CompilerParams()
    if "needs_layout_passes" in pltpu.CompilerParams.__dataclass_fields__:
        cp = dataclasses.replace(cp, needs_layout_passes=False)
    mesh = plsc.VectorSubcoreMesh(core_axis_name="c", subcore_axis_name="s")

    @functools.partial(
        pl.kernel, mesh=mesh, compiler_params=cp,
        out_type=[
            jax.ShapeDtypeStruct((B, _VP), jnp.int32),
            jax.ShapeDtypeStruct((B, _VP), jnp.int32),
        ],
        scratch_types=[
            pltpu.VMEM((_SH,), jnp.int32),
            pltpu.VMEM((_SH,), jnp.int32),
            pltpu.VMEM((_SH,), jnp.int32),
            pltpu.VMEM((_RADIX,), jnp.int32),
            pltpu.VMEM((_RADIX,), jnp.int32),
            pltpu.VMEM((_NS * _RADIX,), jnp.int32),
            pltpu.VMEM_SHARED((_VP,), jnp.int32),
            pltpu.VMEM_SHARED((_VP,), jnp.int32),
            pltpu.VMEM_SHARED((_VP,), jnp.int32),
            pltpu.VMEM_SHARED((_VP,), jnp.int32),
            pltpu.VMEM_SHARED((_NS * _RADIX,), jnp.int32),
            pltpu.SemaphoreType.DMA,
            pltpu.SemaphoreType.DMA,
        ],
    )
    def sortk(keys_hbm, okeys_hbm, ovals_hbm, kl, vl, pos, hist, cursor,
              gridl, ka, va, kb, vb, grid_sh, sem1, sem2):
        cid = lax.axis_index("c")
        sid = lax.axis_index("s")
        base = sid * _SH

        def digits_of(k16, shift):
            d = k16 if shift == 0 else lax.shift_right_logical(k16, shift)
            return jnp.bitwise_and(d, 0xFF)

        def build_hist(shift):
            @pl.loop(0, _RADIX, step=16)
            def _(b):
                hist[pl.ds(b, 16)] = jnp.zeros((16,), jnp.int32)

            @pl.loop(0, _SH, step=16)
            def _(i):
                d = digits_of(kl[pl.ds(i, 16)], shift)
                cnt, last = plsc.scan_count(d)
                cur = plsc.load_gather(hist, [d])
                plsc.store_scatter(hist, [d], cur + cnt, mask=last)

        def compute_cursor():
            pltpu.sync_copy(grid_sh, gridl)

            def chunk(j, carry):
                tot = jnp.zeros((16,), jnp.int32)
                part = jnp.zeros((16,), jnp.int32)
                for t in range(_NS):
                    row = gridl[pl.ds(t * _RADIX + j * 16, 16)]
                    tot = tot + row
                    sel = lax.convert_element_type(t < sid, jnp.int32)
                    part = part + row * lax.broadcast_in_dim(sel, (16,), ())
                excl = plsc.cumsum(tot) - tot
                carryv = lax.broadcast_in_dim(carry, (16,), ())
                cursor[pl.ds(j * 16, 16)] = carryv + excl + part
                return carry + jnp.sum(tot)

            lax.fori_loop(0, _RADIX // 16, chunk, jnp.int32(0))

        def permute(shift, kdst, vdst, first):
            @pl.loop(0, _SH, step=16)
            def _(i):
                d = digits_of(kl[pl.ds(i, 16)], shift)
                cnt, last = plsc.scan_count(d)
                bse = plsc.load_gather(cursor, [d])
                plsc.store_scatter(cursor, [d], bse + cnt, mask=last)
                pos[pl.ds(i, 16)] = bse + cnt - 1
                if first:
                    vl[pl.ds(i, 16)] = base + i + lax.iota(jnp.int32, 16)

            c1 = pltpu.async_copy(kl, kdst.at[pos], sem1)
            c2 = pltpu.async_copy(vl, vdst.at[pos], sem2)
            c1.wait()
            c2.wait()

        @pl.loop(0, rows_per_core)
        def _(r):
            row = cid * rows_per_core + r
            pltpu.sync_copy(keys_hbm.at[row, pl.ds(base, _SH)], kl)
            build_hist(0)
            pltpu.sync_copy(hist, grid_sh.at[pl.ds(sid * _RADIX, _RADIX)])
            plsc.subcore_barrier()
            compute_cursor()
            permute(0, ka, va, first=True)
            plsc.subcore_barrier()
            for shift, ksrc, vsrc, kdst, vdst in (
                    (8, ka, va, kb, vb),
                    (16, kb, vb, ka, va),
                    (24, ka, va, kb, vb)):
                c1 = pltpu.async_copy(ksrc.at[pl.ds(base, _SH)], kl, sem1)
                c2 = pltpu.async_copy(vsrc.at[pl.ds(base, _SH)], vl, sem2)
                c1.wait()
                build_hist(shift)
                pltpu.sync_copy(hist, grid_sh.at[pl.ds(sid * _RADIX, _RADIX)])
                plsc.subcore_barrier()
                compute_cursor()
                c2.wait()
                permute(shift, kdst, vdst, first=False)
                plsc.subcore_barrier()
            c1 = pltpu.async_copy(kb.at[pl.ds(base, _SH)],
                                  okeys_hbm.at[row, pl.ds(base, _SH)], sem1)
            c2 = pltpu.async_copy(vb.at[pl.ds(base, _SH)],
                                  ovals_hbm.at[row, pl.ds(base, _SH)], sem2)
            c1.wait()
            c2.wait()

    return sortk(keys)


def _c1_body(skey_ref, t_ref, s0_ref, esum_ref):
    j = pl.program_id(0)
    col = j * _VT + jax.lax.broadcasted_iota(jnp.int32, skey_ref.shape, 1)
    s = _key_to_logit(skey_ref[...]) / t_ref[...]
    e = jnp.exp(s - s0_ref[...])
    e = jnp.where(col < _V, e, 0.0)
    esum_ref[0, 0, :] = jnp.sum(e, axis=1)


def _c1(skeys, t, s0):
    B = skeys.shape[0]
    nb = _VP // _VT
    return pl.pallas_call(
        _c1_body,
        grid=(nb,),
        in_specs=[
            pl.BlockSpec((B, _VT), lambda i: (0, i)),
            pl.BlockSpec((B, 1), lambda i: (0, 0)),
            pl.BlockSpec((B, 1), lambda i: (0, 0)),
        ],
        out_specs=pl.BlockSpec((1, 1, B), lambda i: (i, 0, 0)),
        out_shape=jax.ShapeDtypeStruct((nb, 1, B), jnp.float32),
    )(skeys, t, s0)[:, 0, :].T


def _cumsum_lanes(x):
    n = x.shape[-1]
    shift = 1
    while shift < n:
        z = jnp.zeros(x.shape[:-1] + (shift,), x.dtype)
        x = x + jnp.concatenate([z, x[..., :-shift]], axis=-1)
        shift *= 2
    return x


def _c2_body(skey_ref, sidx_ref, g_ref, t_ref, s0_ref, z_ref, carry_ref,
             k_ref, p_ref, maxv_ref, tok_ref):
    j = pl.program_id(0)
    col = j * _VT + jax.lax.broadcasted_iota(jnp.int32, skey_ref.shape, 1)
    valid = col < _V
    s = _key_to_logit(skey_ref[...]) / t_ref[...]
    e = jnp.exp(s - s0_ref[...])
    probs = jnp.where(valid, e / z_ref[...], 0.0)
    carry = carry_ref[0, 0, :][:, None]
    cum_excl = carry + _cumsum_lanes(probs) - probs
    mask = (col < k_ref[...]) & (cum_excl < p_ref[...])
    mask = mask | (col == 0)
    mask = mask & valid
    val = jnp.where(mask, s, _NEG_INF) + g_ref[...]
    val = jnp.where(valid, val, _NEG_INF)
    m = jnp.max(val, axis=1, keepdims=True)
    pos = jnp.min(jnp.where(val == m, col, jnp.int32(2**30)), axis=1,
                  keepdims=True)
    lpos = col == pos
    tok = jnp.max(jnp.where(lpos, sidx_ref[...], -1), axis=1, keepdims=True)
    maxv_ref[0, 0, :] = m[:, 0]
    tok_ref[0, 0, :] = tok[:, 0]


def _c2(skeys, sidx, g, t, s0, z, carry, k, p):
    B = skeys.shape[0]
    nb = _VP // _VT
    return pl.pallas_call(
        _c2_body,
        grid=(nb,),
        in_specs=[
            pl.BlockSpec((B, _VT), lambda i: (0, i)),
            pl.BlockSpec((B, _VT), lambda i: (0, i)),
            pl.BlockSpec((B, _VT), lambda i: (0, i)),
            pl.BlockSpec((B, 1), lambda i: (0, 0)),
            pl.BlockSpec((B, 1), lambda i: (0, 0)),
            pl.BlockSpec((B, 1), lambda i: (0, 0)),
            pl.BlockSpec((1, 1, B), lambda i: (i, 0, 0)),
            pl.BlockSpec((B, 1), lambda i: (0, 0)),
            pl.BlockSpec((B, 1), lambda i: (0, 0)),
        ],
        out_specs=[
            pl.BlockSpec((1, 1, B), lambda i: (i, 0, 0)),
            pl.BlockSpec((1, 1, B), lambda i: (i, 0, 0)),
        ],
        out_shape=[
            jax.ShapeDtypeStruct((nb, 1, B), jnp.float32),
            jax.ShapeDtypeStruct((nb, 1, B), jnp.int32),
        ],
    )(skeys, sidx, g, t, s0, z, carry, k, p)


def kernel(hidden_states, temperatures, top_ps, embedding, last_token_indices, top_ks):
    B = temperatures.shape[0]
    h = jnp.take(hidden_states, last_token_indices, axis=0)
    keys = _proj_keys(h, embedding)

    skeys, sidx = _sc_sort(keys)

    t = jnp.where(temperatures < _SAMPLING_EPS, 1.0, temperatures)[:, None]
    kk = jnp.maximum(top_ks, 1)[:, None]
    p = jnp.clip(top_ps, _SAMPLING_EPS, 1.0)[:, None]
    s0 = _key_to_logit(skeys[:, :1]) / t

    esum = _c1(skeys, t, s0)
    z = jnp.sum(esum, axis=1, keepdims=True)
    carry = (jnp.cumsum(esum, axis=1) - esum) / z
    carry = carry.T[:, None, :]

    g = jax.random.gumbel(jax.random.key(42), (B, _V), jnp.float32)
    maxv, tok = _c2(skeys, sidx, g, t, s0, z, carry, kk, p)
    maxv, tok = maxv[:, 0, :].T, tok[:, 0, :].T

    best = jnp.argmax(maxv, axis=1)
    tokens = jnp.take_along_axis(tok, best[:, None], axis=1)[:, 0]
    return tokens


# speedup vs baseline: 1.4293x; 1.4293x over previous
import dataclasses
import functools

import jax
import jax.numpy as jnp
from jax import lax
from jax.experimental import pallas as pl
from jax.experimental.pallas import tpu as pltpu
from jax.experimental.pallas import tpu_sc as plsc

_SAMPLING_EPS = 1e-05
_VT = 2048
_V = 100000
_VP = 100352
_NS = 16
_SH = _VP // _NS
_RADIX = 256
_NEG_INF = float("-inf")


def _proj_keys_body(h_ref, e_ref, key_ref):
    j = pl.program_id(0)
    logits = jax.lax.dot_general(
        h_ref[...], e_ref[...], (((1,), (1,)), ((), ())),
        preferred_element_type=jnp.float32)
    u = jax.lax.bitcast_convert_type(logits + 0.0, jnp.uint32)
    neg = (u >> 31) == 1
    key = jnp.where(neg, u, ~u & jnp.uint32(0x7FFFFFFF))
    col = j * _VT + jax.lax.broadcasted_iota(jnp.int32, key.shape, 1)
    key = jnp.where(col < _V, key, jnp.uint32(0xFFFFFFFF))
    key_ref[...] = jax.lax.bitcast_convert_type(key, jnp.int32)


def _proj_keys(h, emb):
    B, D = h.shape
    grid = (_VP // _VT,)
    return pl.pallas_call(
        _proj_keys_body,
        grid=grid,
        in_specs=[
            pl.BlockSpec((B, D), lambda i: (0, 0)),
            pl.BlockSpec((_VT, D), lambda i: (i, 0)),
        ],
        out_specs=pl.BlockSpec((B, _VT), lambda i: (0, i)),
        out_shape=jax.ShapeDtypeStruct((B, _VP), jnp.int32),
    )(h, emb)


def _key_to_logit(key):
    k = jax.lax.bitcast_convert_type(key, jnp.uint32)
    neg = (k >> 31) == 1
    u = jnp.where(neg, k, ~k & jnp.uint32(0x7FFFFFFF))
    return jax.lax.bitcast_convert_type(u, jnp.float32)


def _sc_sort(keys):
    B = keys.shape[0]
    rows_per_core = B // 2
    cp = pltpu.CompilerParams()
    if "needs_layout_passes" in pltpu.CompilerParams.__dataclass_fields__:
        cp = dataclasses.replace(cp, needs_layout_passes=False)
    mesh = plsc.VectorSubcoreMesh(core_axis_name="c", subcore_axis_name="s")

    @functools.partial(
        pl.kernel, mesh=mesh, compiler_params=cp,
        out_type=[
            jax.ShapeDtypeStruct((B, _VP), jnp.int32),
            jax.ShapeDtypeStruct((B, _VP), jnp.int32),
        ],
        scratch_types=[
            pltpu.VMEM((_SH,), jnp.int32),
            pltpu.VMEM((_SH,), jnp.int32),
            pltpu.VMEM((_SH,), jnp.int32),
            pltpu.VMEM((_RADIX,), jnp.int32),
            pltpu.VMEM((_RADIX,), jnp.int32),
            pltpu.VMEM((_RADIX,), jnp.int32),
            pltpu.VMEM((_RADIX,), jnp.int32),
            pltpu.VMEM((_RADIX,), jnp.int32),
            pltpu.VMEM((_NS * _RADIX,), jnp.int32),
            pltpu.VMEM_SHARED((_VP,), jnp.int32),
            pltpu.VMEM_SHARED((_VP,), jnp.int32),
            pltpu.VMEM_SHARED((_VP,), jnp.int32),
            pltpu.VMEM_SHARED((_VP,), jnp.int32),
            pltpu.VMEM_SHARED((_NS * _RADIX,), jnp.int32),
            pltpu.SemaphoreType.DMA,
            pltpu.SemaphoreType.DMA,
        ],
    )
    def sortk(keys_hbm, okeys_hbm, ovals_hbm, kl, vl, pos, hist, histb,
              histc, cursor, cursorb, gridl, ka, va, kb, vb, grid_sh,
              sem1, sem2):
        cid = lax.axis_index("c")
        sid = lax.axis_index("s")
        base = sid * _SH

        def digits_of(k16, shift):
            d = k16 if shift == 0 else lax.shift_right_logical(k16, shift)
            return jnp.bitwise_and(d, 0xFF)

        half = _SH // 2

        def build_hist(shift):
            @pl.loop(0, _RADIX, step=16)
            def _(b):
                z = jnp.zeros((16,), jnp.int32)
                hist[pl.ds(b, 16)] = z
                histb[pl.ds(b, 16)] = z

            @pl.loop(0, half, step=16)
            def _(i):
                da = digits_of(kl[pl.ds(i, 16)], shift)
                db = digits_of(kl[pl.ds(half + i, 16)], shift)
                ca, la = plsc.scan_count(da)
                cb, lb = plsc.scan_count(db)
                cura = plsc.load_gather(hist, [da])
                curb = plsc.load_gather(histb, [db])
                plsc.store_scatter(hist, [da], cura + ca, mask=la)
                plsc.store_scatter(histb, [db], curb + cb, mask=lb)

            @pl.loop(0, _RADIX, step=16)
            def _(b):
                histc[pl.ds(b, 16)] = hist[pl.ds(b, 16)] + histb[pl.ds(b, 16)]

        def compute_cursor():
            pltpu.sync_copy(grid_sh, gridl)

            def chunk(j, carry):
                tot = jnp.zeros((16,), jnp.int32)
                part = jnp.zeros((16,), jnp.int32)
                for t in range(_NS):
                    row = gridl[pl.ds(t * _RADIX + j * 16, 16)]
                    tot = tot + row
                    sel = lax.convert_element_type(t < sid, jnp.int32)
                    part = part + row * lax.broadcast_in_dim(sel, (16,), ())
                excl = plsc.cumsum(tot) - tot
                carryv = lax.broadcast_in_dim(carry, (16,), ())
                cursor[pl.ds(j * 16, 16)] = carryv + excl + part
                return carry + jnp.sum(tot)

            lax.fori_loop(0, _RADIX // 16, chunk, jnp.int32(0))

            @pl.loop(0, _RADIX, step=16)
            def _(b):
                cursorb[pl.ds(b, 16)] = cursor[pl.ds(b, 16)] + hist[pl.ds(b, 16)]

        def permute(shift, kdst, vdst, first):
            @pl.loop(0, half, step=16)
            def _(i):
                da = digits_of(kl[pl.ds(i, 16)], shift)
                db = digits_of(kl[pl.ds(half + i, 16)], shift)
                ca, la = plsc.scan_count(da)
                cb, lb = plsc.scan_count(db)
                bsa = plsc.load_gather(cursor, [da])
                bsb = plsc.load_gather(cursorb, [db])
                plsc.store_scatter(cursor, [da], bsa + ca, mask=la)
                plsc.store_scatter(cursorb, [db], bsb + cb, mask=lb)
                pos[pl.ds(i, 16)] = bsa + ca - 1
                pos[pl.ds(half + i, 16)] = bsb + cb - 1
                if first:
                    it = lax.iota(jnp.int32, 16)
                    vl[pl.ds(i, 16)] = base + i + it
                    vl[pl.ds(half + i, 16)] = base + half + i + it

            c1 = pltpu.async_copy(kl, kdst.at[pos], sem1)
            c2 = pltpu.async_copy(vl, vdst.at[pos], sem2)
            c1.wait()
            c2.wait()

        @pl.loop(0, rows_per_core)
        def _(r):
            row = cid * rows_per_core + r
            pltpu.sync_copy(keys_hbm.at[row, pl.ds(base, _SH)], kl)
            build_hist(0)
            pltpu.sync_copy(histc, grid_sh.at[pl.ds(sid * _RADIX, _RADIX)])
            plsc.subcore_barrier()
            compute_cursor()
            permute(0, ka, va, first=True)
            plsc.subcore_barrier()
            for shift, ksrc, vsrc, kdst, vdst in (
                    (8, ka, va, kb, vb),
                    (16, kb, vb, ka, va),
                    (24, ka, va, kb, vb)):
                c1 = pltpu.async_copy(ksrc.at[pl.ds(base, _SH)], kl, sem1)
                c2 = pltpu.async_copy(vsrc.at[pl.ds(base, _SH)], vl, sem2)
                c1.wait()
                build_hist(shift)
                pltpu.sync_copy(histc, grid_sh.at[pl.ds(sid * _RADIX, _RADIX)])
                plsc.subcore_barrier()
                compute_cursor()
                c2.wait()
                permute(shift, kdst, vdst, first=False)
                plsc.subcore_barrier()
            c1 = pltpu.async_copy(kb.at[pl.ds(base, _SH)],
                                  okeys_hbm.at[row, pl.ds(base, _SH)], sem1)
            c2 = pltpu.async_copy(vb.at[pl.ds(base, _SH)],
                                  ovals_hbm.at[row, pl.ds(base, _SH)], sem2)
            c1.wait()
            c2.wait()

    return sortk(keys)


def _c1_body(skey_ref, t_ref, s0_ref, esum_ref):
    j = pl.program_id(0)
    col = j * _VT + jax.lax.broadcasted_iota(jnp.int32, skey_ref.shape, 1)
    s = _key_to_logit(skey_ref[...]) / t_ref[...]
    e = jnp.exp(s - s0_ref[...])
    e = jnp.where(col < _V, e, 0.0)
    esum_ref[0, 0, :] = jnp.sum(e, axis=1)


def _c1(skeys, t, s0):
    B = skeys.shape[0]
    nb = _VP // _VT
    return pl.pallas_call(
        _c1_body,
        grid=(nb,),
        in_specs=[
            pl.BlockSpec((B, _VT), lambda i: (0, i)),
            pl.BlockSpec((B, 1), lambda i: (0, 0)),
            pl.BlockSpec((B, 1), lambda i: (0, 0)),
        ],
        out_specs=pl.BlockSpec((1, 1, B), lambda i: (i, 0, 0)),
        out_shape=jax.ShapeDtypeStruct((nb, 1, B), jnp.float32),
    )(skeys, t, s0)[:, 0, :].T


def _cumsum_lanes(x):
    n = x.shape[-1]
    shift = 1
    while shift < n:
        z = jnp.zeros(x.shape[:-1] + (shift,), x.dtype)
        x = x + jnp.concatenate([z, x[..., :-shift]], axis=-1)
        shift *= 2
    return x


def _c2_body(skey_ref, sidx_ref, g_ref, t_ref, s0_ref, z_ref, carry_ref,
             k_ref, p_ref, maxv_ref, tok_ref):
    j = pl.program_id(0)
    col = j * _VT + jax.lax.broadcasted_iota(jnp.int32, skey_ref.shape, 1)
    valid = col < _V
    s = _key_to_logit(skey_ref[...]) / t_ref[...]
    e = jnp.exp(s - s0_ref[...])
    probs = jnp.where(valid, e / z_ref[...], 0.0)
    carry = carry_ref[0, 0, :][:, None]
    cum_excl = carry + _cumsum_lanes(probs) - probs
    mask = (col < k_ref[...]) & (cum_excl < p_ref[...])
    mask = mask | (col == 0)
    mask = mask & valid
    val = jnp.where(mask, s, _NEG_INF) + g_ref[...]
    val = jnp.where(valid, val, _NEG_INF)
    m = jnp.max(val, axis=1, keepdims=True)
    pos = jnp.min(jnp.where(val == m, col, jnp.int32(2**30)), axis=1,
                  keepdims=True)
    lpos = col == pos
    tok = jnp.max(jnp.where(lpos, sidx_ref[...], -1), axis=1, keepdims=True)
    maxv_ref[0, 0, :] = m[:, 0]
    tok_ref[0, 0, :] = tok[:, 0]


def _c2(skeys, sidx, g, t, s0, z, carry, k, p):
    B = skeys.shape[0]
    nb = _VP // _VT
    return pl.pallas_call(
        _c2_body,
        grid=(nb,),
        in_specs=[
            pl.BlockSpec((B, _VT), lambda i: (0, i)),
            pl.BlockSpec((B, _VT), lambda i: (0, i)),
            pl.BlockSpec((B, _VT), lambda i: (0, i)),
            pl.BlockSpec((B, 1), lambda i: (0, 0)),
            pl.BlockSpec((B, 1), lambda i: (0, 0)),
            pl.BlockSpec((B, 1), lambda i: (0, 0)),
            pl.BlockSpec((1, 1, B), lambda i: (i, 0, 0)),
            pl.BlockSpec((B, 1), lambda i: (0, 0)),
            pl.BlockSpec((B, 1), lambda i: (0, 0)),
        ],
        out_specs=[
            pl.BlockSpec((1, 1, B), lambda i: (i, 0, 0)),
            pl.BlockSpec((1, 1, B), lambda i: (i, 0, 0)),
        ],
        out_shape=[
            jax.ShapeDtypeStruct((nb, 1, B), jnp.float32),
            jax.ShapeDtypeStruct((nb, 1, B), jnp.int32),
        ],
    )(skeys, sidx, g, t, s0, z, carry, k, p)


def kernel(hidden_states, temperatures, top_ps, embedding, last_token_indices, top_ks):
    B = temperatures.shape[0]
    h = jnp.take(hidden_states, last_token_indices, axis=0)
    keys = _proj_keys(h, embedding)

    skeys, sidx = _sc_sort(keys)

    t = jnp.where(temperatures < _SAMPLING_EPS, 1.0, temperatures)[:, None]
    kk = jnp.maximum(top_ks, 1)[:, None]
    p = jnp.clip(top_ps, _SAMPLING_EPS, 1.0)[:, None]
    s0 = _key_to_logit(skeys[:, :1]) / t

    esum = _c1(skeys, t, s0)
    z = jnp.sum(esum, axis=1, keepdims=True)
    carry = (jnp.cumsum(esum, axis=1) - esum) / z
    carry = carry.T[:, None, :]

    g = jax.random.gumbel(jax.random.key(42), (B, _V), jnp.float32)
    maxv, tok = _c2(skeys, sidx, g, t, s0, z, carry, kk, p)
    maxv, tok = maxv[:, 0, :].T, tok[:, 0, :].T

    best = jnp.argmax(maxv, axis=1)
    tokens = jnp.take_along_axis(tok, best[:, None], axis=1)[:, 0]
    return tokens


# speedup vs baseline: 1.7416x; 1.2185x over previous
import dataclasses
import functools

import jax
import jax.numpy as jnp
from jax import lax
from jax.experimental import pallas as pl
from jax.experimental.pallas import tpu as pltpu
from jax.experimental.pallas import tpu_sc as plsc

_SAMPLING_EPS = 1e-05
_VT = 2048
_V = 100000
_VP = 100352
_NS = 16
_SH = _VP // _NS
_RADIX = 256
_NEG_INF = float("-inf")


def _proj_keys_body(h_ref, e_ref, key_ref):
    j = pl.program_id(0)
    logits = jax.lax.dot_general(
        h_ref[...], e_ref[...], (((1,), (1,)), ((), ())),
        preferred_element_type=jnp.float32)
    u = jax.lax.bitcast_convert_type(logits + 0.0, jnp.uint32)
    neg = (u >> 31) == 1
    key = jnp.where(neg, u, ~u & jnp.uint32(0x7FFFFFFF))
    col = j * _VT + jax.lax.broadcasted_iota(jnp.int32, key.shape, 1)
    key = jnp.where(col < _V, key, jnp.uint32(0xFFFFFFFF))
    key_ref[...] = jax.lax.bitcast_convert_type(key, jnp.int32)


def _proj_keys(h, emb):
    B, D = h.shape
    grid = (_VP // _VT,)
    return pl.pallas_call(
        _proj_keys_body,
        grid=grid,
        in_specs=[
            pl.BlockSpec((B, D), lambda i: (0, 0)),
            pl.BlockSpec((_VT, D), lambda i: (i, 0)),
        ],
        out_specs=pl.BlockSpec((B, _VT), lambda i: (0, i)),
        out_shape=jax.ShapeDtypeStruct((B, _VP), jnp.int32),
    )(h, emb)


def _key_to_logit(key):
    k = jax.lax.bitcast_convert_type(key, jnp.uint32)
    neg = (k >> 31) == 1
    u = jnp.where(neg, k, ~k & jnp.uint32(0x7FFFFFFF))
    return jax.lax.bitcast_convert_type(u, jnp.float32)


def _sc_sort(keys):
    B = keys.shape[0]
    rows_per_core = B // 2
    cp = pltpu.CompilerParams()
    if "needs_layout_passes" in pltpu.CompilerParams.__dataclass_fields__:
        cp = dataclasses.replace(cp, needs_layout_passes=False)
    mesh = plsc.VectorSubcoreMesh(core_axis_name="c", subcore_axis_name="s")

    W = 4

    @functools.partial(
        pl.kernel, mesh=mesh, compiler_params=cp,
        out_type=[
            jax.ShapeDtypeStruct((B, _VP), jnp.int32),
            jax.ShapeDtypeStruct((B, _VP), jnp.int32),
        ],
        scratch_types=[
            pltpu.VMEM((_SH,), jnp.int32),
            pltpu.VMEM((_SH,), jnp.int32),
            pltpu.VMEM((_SH,), jnp.int32),
        ] + [pltpu.VMEM((_RADIX,), jnp.int32)] * W
        + [pltpu.VMEM((_RADIX,), jnp.int32)]
        + [pltpu.VMEM((_RADIX,), jnp.int32)] * W
        + [
            pltpu.VMEM((_NS * _RADIX,), jnp.int32),
            pltpu.VMEM_SHARED((_VP,), jnp.int32),
            pltpu.VMEM_SHARED((_VP,), jnp.int32),
            pltpu.VMEM_SHARED((_VP,), jnp.int32),
            pltpu.VMEM_SHARED((_VP,), jnp.int32),
            pltpu.VMEM_SHARED((_NS * _RADIX,), jnp.int32),
            pltpu.SemaphoreType.DMA,
            pltpu.SemaphoreType.DMA,
        ],
    )
    def sortk(keys_hbm, okeys_hbm, ovals_hbm, kl, vl, pos, *rest):
        hists = rest[:W]
        histc = rest[W]
        cursors = rest[W + 1:2 * W + 1]
        (gridl, ka, va, kb, vb, grid_sh, sem1, sem2) = rest[2 * W + 1:]
        cid = lax.axis_index("c")
        sid = lax.axis_index("s")
        base = sid * _SH
        part_len = _SH // W
        offs = [w * part_len for w in range(W)]

        def digits_of(k16, shift):
            d = k16 if shift == 0 else lax.shift_right_logical(k16, shift)
            return jnp.bitwise_and(d, 0xFF)

        def build_hist(shift):
            @pl.loop(0, _RADIX, step=16)
            def _(b):
                z = jnp.zeros((16,), jnp.int32)
                for h in hists:
                    h[pl.ds(b, 16)] = z

            @pl.loop(0, part_len, step=16)
            def _(i):
                ds = [digits_of(kl[pl.ds(offs[w] + i, 16)], shift)
                      for w in range(W)]
                cl = [plsc.scan_count(d) for d in ds]
                cur = [plsc.load_gather(hists[w], [ds[w]]) for w in range(W)]
                for w in range(W):
                    plsc.store_scatter(hists[w], [ds[w]], cur[w] + cl[w][0],
                                       mask=cl[w][1])

            @pl.loop(0, _RADIX, step=16)
            def _(b):
                acc = hists[0][pl.ds(b, 16)]
                for h in hists[1:]:
                    acc = acc + h[pl.ds(b, 16)]
                histc[pl.ds(b, 16)] = acc

        def compute_cursor():
            pltpu.sync_copy(grid_sh, gridl)

            def chunk(j, carry):
                tot = jnp.zeros((16,), jnp.int32)
                part = jnp.zeros((16,), jnp.int32)
                for t in range(_NS):
                    row = gridl[pl.ds(t * _RADIX + j * 16, 16)]
                    tot = tot + row
                    sel = lax.convert_element_type(t < sid, jnp.int32)
                    part = part + row * lax.broadcast_in_dim(sel, (16,), ())
                excl = plsc.cumsum(tot) - tot
                carryv = lax.broadcast_in_dim(carry, (16,), ())
                cursors[0][pl.ds(j * 16, 16)] = carryv + excl + part
                return carry + jnp.sum(tot)

            lax.fori_loop(0, _RADIX // 16, chunk, jnp.int32(0))

            @pl.loop(0, _RADIX, step=16)
            def _(b):
                acc = cursors[0][pl.ds(b, 16)]
                for w in range(1, W):
                    acc = acc + hists[w - 1][pl.ds(b, 16)]
                    cursors[w][pl.ds(b, 16)] = acc

        def permute(shift, kdst, vdst, first):
            @pl.loop(0, part_len, step=16)
            def _(i):
                ds = [digits_of(kl[pl.ds(offs[w] + i, 16)], shift)
                      for w in range(W)]
                cl = [plsc.scan_count(d) for d in ds]
                bs = [plsc.load_gather(cursors[w], [ds[w]]) for w in range(W)]
                for w in range(W):
                    plsc.store_scatter(cursors[w], [ds[w]],
                                       bs[w] + cl[w][0], mask=cl[w][1])
                for w in range(W):
                    pos[pl.ds(offs[w] + i, 16)] = bs[w] + cl[w][0] - 1
                if first:
                    it = lax.iota(jnp.int32, 16)
                    for w in range(W):
                        vl[pl.ds(offs[w] + i, 16)] = base + offs[w] + i + it

            c1 = pltpu.async_copy(kl, kdst.at[pos], sem1)
            c2 = pltpu.async_copy(vl, vdst.at[pos], sem2)
            c1.wait()
            c2.wait()

        @pl.loop(0, rows_per_core)
        def _(r):
            row = cid * rows_per_core + r
            pltpu.sync_copy(keys_hbm.at[row, pl.ds(base, _SH)], kl)
            build_hist(0)
            pltpu.sync_copy(histc, grid_sh.at[pl.ds(sid * _RADIX, _RADIX)])
            plsc.subcore_barrier()
            compute_cursor()
            permute(0, ka, va, first=True)
            plsc.subcore_barrier()
            for shift, ksrc, vsrc, kdst, vdst in (
                    (8, ka, va, kb, vb),
                    (16, kb, vb, ka, va),
                    (24, ka, va, kb, vb)):
                c1 = pltpu.async_copy(ksrc.at[pl.ds(base, _SH)], kl, sem1)
                c2 = pltpu.async_copy(vsrc.at[pl.ds(base, _SH)], vl, sem2)
                c1.wait()
                build_hist(shift)
                pltpu.sync_copy(histc, grid_sh.at[pl.ds(sid * _RADIX, _RADIX)])
                plsc.subcore_barrier()
                compute_cursor()
                c2.wait()
                permute(shift, kdst, vdst, first=False)
                plsc.subcore_barrier()
            c1 = pltpu.async_copy(kb.at[pl.ds(base, _SH)],
                                  okeys_hbm.at[row, pl.ds(base, _SH)], sem1)
            c2 = pltpu.async_copy(vb.at[pl.ds(base, _SH)],
                                  ovals_hbm.at[row, pl.ds(base, _SH)], sem2)
            c1.wait()
            c2.wait()

    return sortk(keys)


def _c1_body(skey_ref, t_ref, s0_ref, esum_ref):
    j = pl.program_id(0)
    col = j * _VT + jax.lax.broadcasted_iota(jnp.int32, skey_ref.shape, 1)
    s = _key_to_logit(skey_ref[...]) / t_ref[...]
    e = jnp.exp(s - s0_ref[...])
    e = jnp.where(col < _V, e, 0.0)
    esum_ref[0, 0, :] = jnp.sum(e, axis=1)


def _c1(skeys, t, s0):
    B = skeys.shape[0]
    nb = _VP // _VT
    return pl.pallas_call(
        _c1_body,
        grid=(nb,),
        in_specs=[
            pl.BlockSpec((B, _VT), lambda i: (0, i)),
            pl.BlockSpec((B, 1), lambda i: (0, 0)),
            pl.BlockSpec((B, 1), lambda i: (0, 0)),
        ],
        out_specs=pl.BlockSpec((1, 1, B), lambda i: (i, 0, 0)),
        out_shape=jax.ShapeDtypeStruct((nb, 1, B), jnp.float32),
    )(skeys, t, s0)[:, 0, :].T


def _cumsum_lanes(x):
    n = x.shape[-1]
    shift = 1
    while shift < n:
        z = jnp.zeros(x.shape[:-1] + (shift,), x.dtype)
        x = x + jnp.concatenate([z, x[..., :-shift]], axis=-1)
        shift *= 2
    return x


def _c2_body(skey_ref, sidx_ref, g_ref, t_ref, s0_ref, z_ref, carry_ref,
             k_ref, p_ref, maxv_ref, tok_ref):
    j = pl.program_id(0)
    col = j * _VT + jax.lax.broadcasted_iota(jnp.int32, skey_ref.shape, 1)
    valid = col < _V
    s = _key_to_logit(skey_ref[...]) / t_ref[...]
    e = jnp.exp(s - s0_ref[...])
    probs = jnp.where(valid, e / z_ref[...], 0.0)
    carry = carry_ref[0, 0, :][:, None]
    cum_excl = carry + _cumsum_lanes(probs) - probs
    mask = (col < k_ref[...]) & (cum_excl < p_ref[...])
    mask = mask | (col == 0)
    mask = mask & valid
    val = jnp.where(mask, s, _NEG_INF) + g_ref[...]
    val = jnp.where(valid, val, _NEG_INF)
    m = jnp.max(val, axis=1, keepdims=True)
    pos = jnp.min(jnp.where(val == m, col, jnp.int32(2**30)), axis=1,
                  keepdims=True)
    lpos = col == pos
    tok = jnp.max(jnp.where(lpos, sidx_ref[...], -1), axis=1, keepdims=True)
    maxv_ref[0, 0, :] = m[:, 0]
    tok_ref[0, 0, :] = tok[:, 0]


def _c2(skeys, sidx, g, t, s0, z, carry, k, p):
    B = skeys.shape[0]
    nb = _VP // _VT
    return pl.pallas_call(
        _c2_body,
        grid=(nb,),
        in_specs=[
            pl.BlockSpec((B, _VT), lambda i: (0, i)),
            pl.BlockSpec((B, _VT), lambda i: (0, i)),
            pl.BlockSpec((B, _VT), lambda i: (0, i)),
            pl.BlockSpec((B, 1), lambda i: (0, 0)),
            pl.BlockSpec((B, 1), lambda i: (0, 0)),
            pl.BlockSpec((B, 1), lambda i: (0, 0)),
            pl.BlockSpec((1, 1, B), lambda i: (i, 0, 0)),
            pl.BlockSpec((B, 1), lambda i: (0, 0)),
            pl.BlockSpec((B, 1), lambda i: (0, 0)),
        ],
        out_specs=[
            pl.BlockSpec((1, 1, B), lambda i: (i, 0, 0)),
            pl.BlockSpec((1, 1, B), lambda i: (i, 0, 0)),
        ],
        out_shape=[
            jax.ShapeDtypeStruct((nb, 1, B), jnp.float32),
            jax.ShapeDtypeStruct((nb, 1, B), jnp.int32),
        ],
    )(skeys, sidx, g, t, s0, z, carry, k, p)


def kernel(hidden_states, temperatures, top_ps, embedding, last_token_indices, top_ks):
    B = temperatures.shape[0]
    h = jnp.take(hidden_states, last_token_indices, axis=0)
    keys = _proj_keys(h, embedding)

    skeys, sidx = _sc_sort(keys)

    t = jnp.where(temperatures < _SAMPLING_EPS, 1.0, temperatures)[:, None]
    kk = jnp.maximum(top_ks, 1)[:, None]
    p = jnp.clip(top_ps, _SAMPLING_EPS, 1.0)[:, None]
    s0 = _key_to_logit(skeys[:, :1]) / t

    esum = _c1(skeys, t, s0)
    z = jnp.sum(esum, axis=1, keepdims=True)
    carry = (jnp.cumsum(esum, axis=1) - esum) / z
    carry = carry.T[:, None, :]

    g = jax.random.gumbel(jax.random.key(42), (B, _V), jnp.float32)
    maxv, tok = _c2(skeys, sidx, g, t, s0, z, carry, kk, p)
    maxv, tok = maxv[:, 0, :].T, tok[:, 0, :].T

    best = jnp.argmax(maxv, axis=1)
    tokens = jnp.take_along_axis(tok, best[:, None], axis=1)[:, 0]
    return tokens


# speedup vs baseline: 1.9921x; 1.1438x over previous
import dataclasses
import functools

import jax
import jax.numpy as jnp
from jax import lax
from jax.experimental import pallas as pl
from jax.experimental.pallas import tpu as pltpu
from jax.experimental.pallas import tpu_sc as plsc

_SAMPLING_EPS = 1e-05
_VT = 2048
_V = 100000
_VP = 100352
_NS = 16
_SH = _VP // _NS
_RADIX = 256
_NEG_INF = float("-inf")


def _proj_keys_body(h_ref, e_ref, key_ref):
    j = pl.program_id(0)
    logits = jax.lax.dot_general(
        h_ref[...], e_ref[...], (((1,), (1,)), ((), ())),
        preferred_element_type=jnp.float32)
    u = jax.lax.bitcast_convert_type(logits + 0.0, jnp.uint32)
    neg = (u >> 31) == 1
    key = jnp.where(neg, u, ~u & jnp.uint32(0x7FFFFFFF))
    col = j * _VT + jax.lax.broadcasted_iota(jnp.int32, key.shape, 1)
    key = jnp.where(col < _V, key, jnp.uint32(0xFFFFFFFF))
    key_ref[...] = jax.lax.bitcast_convert_type(key, jnp.int32)


def _proj_keys(h, emb):
    B, D = h.shape
    grid = (_VP // _VT,)
    return pl.pallas_call(
        _proj_keys_body,
        grid=grid,
        in_specs=[
            pl.BlockSpec((B, D), lambda i: (0, 0)),
            pl.BlockSpec((_VT, D), lambda i: (i, 0)),
        ],
        out_specs=pl.BlockSpec((B, _VT), lambda i: (0, i)),
        out_shape=jax.ShapeDtypeStruct((B, _VP), jnp.int32),
    )(h, emb)


def _key_to_logit(key):
    k = jax.lax.bitcast_convert_type(key, jnp.uint32)
    neg = (k >> 31) == 1
    u = jnp.where(neg, k, ~k & jnp.uint32(0x7FFFFFFF))
    return jax.lax.bitcast_convert_type(u, jnp.float32)


def _sc_sort(keys):
    B = keys.shape[0]
    rows_per_core = B // 2
    cp = pltpu.CompilerParams()
    if "needs_layout_passes" in pltpu.CompilerParams.__dataclass_fields__:
        cp = dataclasses.replace(cp, needs_layout_passes=False)
    mesh = plsc.VectorSubcoreMesh(core_axis_name="c", subcore_axis_name="s")

    W = 8

    @functools.partial(
        pl.kernel, mesh=mesh, compiler_params=cp,
        out_type=[
            jax.ShapeDtypeStruct((B, _VP), jnp.int32),
            jax.ShapeDtypeStruct((B, _VP), jnp.int32),
        ],
        scratch_types=[
            pltpu.VMEM((_SH,), jnp.int32),
            pltpu.VMEM((_SH,), jnp.int32),
            pltpu.VMEM((_SH,), jnp.int32),
        ] + [pltpu.VMEM((_RADIX,), jnp.int32)] * W
        + [pltpu.VMEM((_RADIX,), jnp.int32)]
        + [pltpu.VMEM((_RADIX,), jnp.int32)] * W
        + [
            pltpu.VMEM((_NS * _RADIX,), jnp.int32),
            pltpu.VMEM_SHARED((_VP,), jnp.int32),
            pltpu.VMEM_SHARED((_VP,), jnp.int32),
            pltpu.VMEM_SHARED((_VP,), jnp.int32),
            pltpu.VMEM_SHARED((_VP,), jnp.int32),
            pltpu.VMEM_SHARED((_NS * _RADIX,), jnp.int32),
            pltpu.SemaphoreType.DMA,
            pltpu.SemaphoreType.DMA,
        ],
    )
    def sortk(keys_hbm, okeys_hbm, ovals_hbm, kl, vl, pos, *rest):
        hists = rest[:W]
        histc = rest[W]
        cursors = rest[W + 1:2 * W + 1]
        (gridl, ka, va, kb, vb, grid_sh, sem1, sem2) = rest[2 * W + 1:]
        cid = lax.axis_index("c")
        sid = lax.axis_index("s")
        base = sid * _SH
        part_len = _SH // W
        offs = [w * part_len for w in range(W)]

        def digits_of(k16, shift):
            d = k16 if shift == 0 else lax.shift_right_logical(k16, shift)
            return jnp.bitwise_and(d, 0xFF)

        def build_hist(shift):
            @pl.loop(0, _RADIX, step=16)
            def _(b):
                z = jnp.zeros((16,), jnp.int32)
                for h in hists:
                    h[pl.ds(b, 16)] = z

            @pl.loop(0, part_len, step=16)
            def _(i):
                ds = [digits_of(kl[pl.ds(offs[w] + i, 16)], shift)
                      for w in range(W)]
                cl = [plsc.scan_count(d) for d in ds]
                cur = [plsc.load_gather(hists[w], [ds[w]]) for w in range(W)]
                for w in range(W):
                    plsc.store_scatter(hists[w], [ds[w]], cur[w] + cl[w][0],
                                       mask=cl[w][1])

            @pl.loop(0, _RADIX, step=16)
            def _(b):
                acc = hists[0][pl.ds(b, 16)]
                for h in hists[1:]:
                    acc = acc + h[pl.ds(b, 16)]
                histc[pl.ds(b, 16)] = acc

        def compute_cursor():
            pltpu.sync_copy(grid_sh, gridl)

            def chunk(j, carry):
                tot = jnp.zeros((16,), jnp.int32)
                part = jnp.zeros((16,), jnp.int32)
                for t in range(_NS):
                    row = gridl[pl.ds(t * _RADIX + j * 16, 16)]
                    tot = tot + row
                    sel = lax.convert_element_type(t < sid, jnp.int32)
                    part = part + row * lax.broadcast_in_dim(sel, (16,), ())
                excl = plsc.cumsum(tot) - tot
                carryv = lax.broadcast_in_dim(carry, (16,), ())
                cursors[0][pl.ds(j * 16, 16)] = carryv + excl + part
                return carry + jnp.sum(tot)

            lax.fori_loop(0, _RADIX // 16, chunk, jnp.int32(0))

            @pl.loop(0, _RADIX, step=16)
            def _(b):
                acc = cursors[0][pl.ds(b, 16)]
                for w in range(1, W):
                    acc = acc + hists[w - 1][pl.ds(b, 16)]
                    cursors[w][pl.ds(b, 16)] = acc

        def permute(shift, kdst, vdst, first):
            @pl.loop(0, part_len, step=16)
            def _(i):
                ds = [digits_of(kl[pl.ds(offs[w] + i, 16)], shift)
                      for w in range(W)]
                cl = [plsc.scan_count(d) for d in ds]
                bs = [plsc.load_gather(cursors[w], [ds[w]]) for w in range(W)]
                for w in range(W):
                    plsc.store_scatter(cursors[w], [ds[w]],
                                       bs[w] + cl[w][0], mask=cl[w][1])
                for w in range(W):
                    pos[pl.ds(offs[w] + i, 16)] = bs[w] + cl[w][0] - 1
                if first:
                    it = lax.iota(jnp.int32, 16)
                    for w in range(W):
                        vl[pl.ds(offs[w] + i, 16)] = base + offs[w] + i + it

            c1 = pltpu.async_copy(kl, kdst.at[pos], sem1)
            c2 = pltpu.async_copy(vl, vdst.at[pos], sem2)
            c1.wait()
            c2.wait()

        @pl.loop(0, rows_per_core)
        def _(r):
            row = cid * rows_per_core + r
            pltpu.sync_copy(keys_hbm.at[row, pl.ds(base, _SH)], kl)
            build_hist(0)
            pltpu.sync_copy(histc, grid_sh.at[pl.ds(sid * _RADIX, _RADIX)])
            plsc.subcore_barrier()
            compute_cursor()
            permute(0, ka, va, first=True)
            plsc.subcore_barrier()
            for shift, ksrc, vsrc, kdst, vdst in (
                    (8, ka, va, kb, vb),
                    (16, kb, vb, ka, va),
                    (24, ka, va, kb, vb)):
                c1 = pltpu.async_copy(ksrc.at[pl.ds(base, _SH)], kl, sem1)
                c2 = pltpu.async_copy(vsrc.at[pl.ds(base, _SH)], vl, sem2)
                c1.wait()
                build_hist(shift)
                pltpu.sync_copy(histc, grid_sh.at[pl.ds(sid * _RADIX, _RADIX)])
                plsc.subcore_barrier()
                compute_cursor()
                c2.wait()
                permute(shift, kdst, vdst, first=False)
                plsc.subcore_barrier()
            c1 = pltpu.async_copy(kb.at[pl.ds(base, _SH)],
                                  okeys_hbm.at[row, pl.ds(base, _SH)], sem1)
            c2 = pltpu.async_copy(vb.at[pl.ds(base, _SH)],
                                  ovals_hbm.at[row, pl.ds(base, _SH)], sem2)
            c1.wait()
            c2.wait()

    return sortk(keys)


def _c1_body(skey_ref, t_ref, s0_ref, esum_ref):
    j = pl.program_id(0)
    col = j * _VT + jax.lax.broadcasted_iota(jnp.int32, skey_ref.shape, 1)
    s = _key_to_logit(skey_ref[...]) / t_ref[...]
    e = jnp.exp(s - s0_ref[...])
    e = jnp.where(col < _V, e, 0.0)
    esum_ref[0, 0, :] = jnp.sum(e, axis=1)


def _c1(skeys, t, s0):
    B = skeys.shape[0]
    nb = _VP // _VT
    return pl.pallas_call(
        _c1_body,
        grid=(nb,),
        in_specs=[
            pl.BlockSpec((B, _VT), lambda i: (0, i)),
            pl.BlockSpec((B, 1), lambda i: (0, 0)),
            pl.BlockSpec((B, 1), lambda i: (0, 0)),
        ],
        out_specs=pl.BlockSpec((1, 1, B), lambda i: (i, 0, 0)),
        out_shape=jax.ShapeDtypeStruct((nb, 1, B), jnp.float32),
    )(skeys, t, s0)[:, 0, :].T


def _cumsum_lanes(x):
    n = x.shape[-1]
    shift = 1
    while shift < n:
        z = jnp.zeros(x.shape[:-1] + (shift,), x.dtype)
        x = x + jnp.concatenate([z, x[..., :-shift]], axis=-1)
        shift *= 2
    return x


def _c2_body(skey_ref, sidx_ref, g_ref, t_ref, s0_ref, z_ref, carry_ref,
             k_ref, p_ref, maxv_ref, tok_ref):
    j = pl.program_id(0)
    col = j * _VT + jax.lax.broadcasted_iota(jnp.int32, skey_ref.shape, 1)
    valid = col < _V
    s = _key_to_logit(skey_ref[...]) / t_ref[...]
    e = jnp.exp(s - s0_ref[...])
    probs = jnp.where(valid, e / z_ref[...], 0.0)
    carry = carry_ref[0, 0, :][:, None]
    cum_excl = carry + _cumsum_lanes(probs) - probs
    mask = (col < k_ref[...]) & (cum_excl < p_ref[...])
    mask = mask | (col == 0)
    mask = mask & valid
    val = jnp.where(mask, s, _NEG_INF) + g_ref[...]
    val = jnp.where(valid, val, _NEG_INF)
    m = jnp.max(val, axis=1, keepdims=True)
    pos = jnp.min(jnp.where(val == m, col, jnp.int32(2**30)), axis=1,
                  keepdims=True)
    lpos = col == pos
    tok = jnp.max(jnp.where(lpos, sidx_ref[...], -1), axis=1, keepdims=True)
    maxv_ref[0, 0, :] = m[:, 0]
    tok_ref[0, 0, :] = tok[:, 0]


def _c2(skeys, sidx, g, t, s0, z, carry, k, p):
    B = skeys.shape[0]
    nb = _VP // _VT
    return pl.pallas_call(
        _c2_body,
        grid=(nb,),
        in_specs=[
            pl.BlockSpec((B, _VT), lambda i: (0, i)),
            pl.BlockSpec((B, _VT), lambda i: (0, i)),
            pl.BlockSpec((B, _VT), lambda i: (0, i)),
            pl.BlockSpec((B, 1), lambda i: (0, 0)),
            pl.BlockSpec((B, 1), lambda i: (0, 0)),
            pl.BlockSpec((B, 1), lambda i: (0, 0)),
            pl.BlockSpec((1, 1, B), lambda i: (i, 0, 0)),
            pl.BlockSpec((B, 1), lambda i: (0, 0)),
            pl.BlockSpec((B, 1), lambda i: (0, 0)),
        ],
        out_specs=[
            pl.BlockSpec((1, 1, B), lambda i: (i, 0, 0)),
            pl.BlockSpec((1, 1, B), lambda i: (i, 0, 0)),
        ],
        out_shape=[
            jax.ShapeDtypeStruct((nb, 1, B), jnp.float32),
            jax.ShapeDtypeStruct((nb, 1, B), jnp.int32),
        ],
    )(skeys, sidx, g, t, s0, z, carry, k, p)


def kernel(hidden_states, temperatures, top_ps, embedding, last_token_indices, top_ks):
    B = temperatures.shape[0]
    h = jnp.take(hidden_states, last_token_indices, axis=0)
    keys = _proj_keys(h, embedding)

    skeys, sidx = _sc_sort(keys)

    t = jnp.where(temperatures < _SAMPLING_EPS, 1.0, temperatures)[:, None]
    kk = jnp.maximum(top_ks, 1)[:, None]
    p = jnp.clip(top_ps, _SAMPLING_EPS, 1.0)[:, None]
    s0 = _key_to_logit(skeys[:, :1]) / t

    esum = _c1(skeys, t, s0)
    z = jnp.sum(esum, axis=1, keepdims=True)
    carry = (jnp.cumsum(esum, axis=1) - esum) / z
    carry = carry.T[:, None, :]

    g = jax.random.gumbel(jax.random.key(42), (B, _V), jnp.float32)
    maxv, tok = _c2(skeys, sidx, g, t, s0, z, carry, kk, p)
    maxv, tok = maxv[:, 0, :].T, tok[:, 0, :].T

    best = jnp.argmax(maxv, axis=1)
    tokens = jnp.take_along_axis(tok, best[:, None], axis=1)[:, 0]
    return tokens


# speedup vs baseline: 2.0436x; 1.0259x over previous
import dataclasses
import functools

import jax
import jax.numpy as jnp
from jax import lax
from jax.experimental import pallas as pl
from jax.experimental.pallas import tpu as pltpu
from jax.experimental.pallas import tpu_sc as plsc

_SAMPLING_EPS = 1e-05
_VT = 2048
_V = 100000
_VP = 100352
_NS = 16
_SH = _VP // _NS
_RADIX = 256
_NEG_INF = float("-inf")


def _proj_keys_body(h_ref, e_ref, key_ref):
    j = pl.program_id(0)
    logits = jax.lax.dot_general(
        h_ref[...], e_ref[...], (((1,), (1,)), ((), ())),
        preferred_element_type=jnp.float32)
    u = jax.lax.bitcast_convert_type(logits + 0.0, jnp.uint32)
    neg = (u >> 31) == 1
    key = jnp.where(neg, u, ~u & jnp.uint32(0x7FFFFFFF))
    col = j * _VT + jax.lax.broadcasted_iota(jnp.int32, key.shape, 1)
    key = jnp.where(col < _V, key, jnp.uint32(0xFFFFFFFF))
    key_ref[...] = jax.lax.bitcast_convert_type(key, jnp.int32)


def _proj_keys(h, emb):
    B, D = h.shape
    grid = (_VP // _VT,)
    return pl.pallas_call(
        _proj_keys_body,
        grid=grid,
        in_specs=[
            pl.BlockSpec((B, D), lambda i: (0, 0)),
            pl.BlockSpec((_VT, D), lambda i: (i, 0)),
        ],
        out_specs=pl.BlockSpec((B, _VT), lambda i: (0, i)),
        out_shape=jax.ShapeDtypeStruct((B, _VP), jnp.int32),
    )(h, emb)


def _key_to_logit(key):
    k = jax.lax.bitcast_convert_type(key, jnp.uint32)
    neg = (k >> 31) == 1
    u = jnp.where(neg, k, ~k & jnp.uint32(0x7FFFFFFF))
    return jax.lax.bitcast_convert_type(u, jnp.float32)


def _sc_sort(keys):
    B = keys.shape[0]
    rows_per_core = B // 2
    cp = pltpu.CompilerParams()
    if "needs_layout_passes" in pltpu.CompilerParams.__dataclass_fields__:
        cp = dataclasses.replace(cp, needs_layout_passes=False)
    mesh = plsc.VectorSubcoreMesh(core_axis_name="c", subcore_axis_name="s")

    W = 14

    @functools.partial(
        pl.kernel, mesh=mesh, compiler_params=cp,
        out_type=[
            jax.ShapeDtypeStruct((B, _VP), jnp.int32),
            jax.ShapeDtypeStruct((B, _VP), jnp.int32),
        ],
        scratch_types=[
            pltpu.VMEM((_SH,), jnp.int32),
            pltpu.VMEM((_SH,), jnp.int32),
            pltpu.VMEM((_SH,), jnp.int32),
        ] + [pltpu.VMEM((_RADIX,), jnp.int32)] * W
        + [pltpu.VMEM((_RADIX,), jnp.int32)]
        + [pltpu.VMEM((_RADIX,), jnp.int32)] * W
        + [
            pltpu.VMEM((_NS * _RADIX,), jnp.int32),
            pltpu.VMEM_SHARED((_VP,), jnp.int32),
            pltpu.VMEM_SHARED((_VP,), jnp.int32),
            pltpu.VMEM_SHARED((_VP,), jnp.int32),
            pltpu.VMEM_SHARED((_VP,), jnp.int32),
            pltpu.VMEM_SHARED((_NS * _RADIX,), jnp.int32),
            pltpu.SemaphoreType.DMA,
            pltpu.SemaphoreType.DMA,
        ],
    )
    def sortk(keys_hbm, okeys_hbm, ovals_hbm, kl, vl, pos, *rest):
        hists = rest[:W]
        histc = rest[W]
        cursors = rest[W + 1:2 * W + 1]
        (gridl, ka, va, kb, vb, grid_sh, sem1, sem2) = rest[2 * W + 1:]
        cid = lax.axis_index("c")
        sid = lax.axis_index("s")
        base = sid * _SH
        part_len = _SH // W
        offs = [w * part_len for w in range(W)]

        def digits_of(k16, shift):
            d = k16 if shift == 0 else lax.shift_right_logical(k16, shift)
            return jnp.bitwise_and(d, 0xFF)

        def build_hist(shift):
            @pl.loop(0, _RADIX, step=16)
            def _(b):
                z = jnp.zeros((16,), jnp.int32)
                for h in hists:
                    h[pl.ds(b, 16)] = z

            @pl.loop(0, part_len, step=16)
            def _(i):
                ds = [digits_of(kl[pl.ds(offs[w] + i, 16)], shift)
                      for w in range(W)]
                cl = [plsc.scan_count(d) for d in ds]
                cur = [plsc.load_gather(hists[w], [ds[w]]) for w in range(W)]
                for w in range(W):
                    plsc.store_scatter(hists[w], [ds[w]], cur[w] + cl[w][0],
                                       mask=cl[w][1])

            @pl.loop(0, _RADIX, step=16)
            def _(b):
                acc = hists[0][pl.ds(b, 16)]
                for h in hists[1:]:
                    acc = acc + h[pl.ds(b, 16)]
                histc[pl.ds(b, 16)] = acc

        def compute_cursor():
            pltpu.sync_copy(grid_sh, gridl)

            def chunk(j, carry):
                tot = jnp.zeros((16,), jnp.int32)
                part = jnp.zeros((16,), jnp.int32)
                for t in range(_NS):
                    row = gridl[pl.ds(t * _RADIX + j * 16, 16)]
                    tot = tot + row
                    sel = lax.convert_element_type(t < sid, jnp.int32)
                    part = part + row * lax.broadcast_in_dim(sel, (16,), ())
                excl = plsc.cumsum(tot) - tot
                carryv = lax.broadcast_in_dim(carry, (16,), ())
                cursors[0][pl.ds(j * 16, 16)] = carryv + excl + part
                return carry + jnp.sum(tot)

            lax.fori_loop(0, _RADIX // 16, chunk, jnp.int32(0))

            @pl.loop(0, _RADIX, step=16)
            def _(b):
                acc = cursors[0][pl.ds(b, 16)]
                for w in range(1, W):
                    acc = acc + hists[w - 1][pl.ds(b, 16)]
                    cursors[w][pl.ds(b, 16)] = acc

        def permute(shift, kdst, vdst, first):
            @pl.loop(0, part_len, step=16)
            def _(i):
                ds = [digits_of(kl[pl.ds(offs[w] + i, 16)], shift)
                      for w in range(W)]
                cl = [plsc.scan_count(d) for d in ds]
                bs = [plsc.load_gather(cursors[w], [ds[w]]) for w in range(W)]
                for w in range(W):
                    plsc.store_scatter(cursors[w], [ds[w]],
                                       bs[w] + cl[w][0], mask=cl[w][1])
                for w in range(W):
                    pos[pl.ds(offs[w] + i, 16)] = bs[w] + cl[w][0] - 1
                if first:
                    it = lax.iota(jnp.int32, 16)
                    for w in range(W):
                        vl[pl.ds(offs[w] + i, 16)] = base + offs[w] + i + it

            c1 = pltpu.async_copy(kl, kdst.at[pos], sem1)
            c2 = pltpu.async_copy(vl, vdst.at[pos], sem2)
            c1.wait()
            c2.wait()

        @pl.loop(0, rows_per_core)
        def _(r):
            row = cid * rows_per_core + r
            pltpu.sync_copy(keys_hbm.at[row, pl.ds(base, _SH)], kl)
            build_hist(0)
            pltpu.sync_copy(histc, grid_sh.at[pl.ds(sid * _RADIX, _RADIX)])
            plsc.subcore_barrier()
            compute_cursor()
            permute(0, ka, va, first=True)
            plsc.subcore_barrier()
            for shift, ksrc, vsrc, kdst, vdst in (
                    (8, ka, va, kb, vb),
                    (16, kb, vb, ka, va),
                    (24, ka, va, kb, vb)):
                c1 = pltpu.async_copy(ksrc.at[pl.ds(base, _SH)], kl, sem1)
                c2 = pltpu.async_copy(vsrc.at[pl.ds(base, _SH)], vl, sem2)
                c1.wait()
                build_hist(shift)
                pltpu.sync_copy(histc, grid_sh.at[pl.ds(sid * _RADIX, _RADIX)])
                plsc.subcore_barrier()
                compute_cursor()
                c2.wait()
                permute(shift, kdst, vdst, first=False)
                plsc.subcore_barrier()
            c1 = pltpu.async_copy(kb.at[pl.ds(base, _SH)],
                                  okeys_hbm.at[row, pl.ds(base, _SH)], sem1)
            c2 = pltpu.async_copy(vb.at[pl.ds(base, _SH)],
                                  ovals_hbm.at[row, pl.ds(base, _SH)], sem2)
            c1.wait()
            c2.wait()

    return sortk(keys)


def _c1_body(skey_ref, t_ref, s0_ref, esum_ref):
    j = pl.program_id(0)
    col = j * _VT + jax.lax.broadcasted_iota(jnp.int32, skey_ref.shape, 1)
    s = _key_to_logit(skey_ref[...]) / t_ref[...]
    e = jnp.exp(s - s0_ref[...])
    e = jnp.where(col < _V, e, 0.0)
    esum_ref[0, 0, :] = jnp.sum(e, axis=1)


def _c1(skeys, t, s0):
    B = skeys.shape[0]
    nb = _VP // _VT
    return pl.pallas_call(
        _c1_body,
        grid=(nb,),
        in_specs=[
            pl.BlockSpec((B, _VT), lambda i: (0, i)),
            pl.BlockSpec((B, 1), lambda i: (0, 0)),
            pl.BlockSpec((B, 1), lambda i: (0, 0)),
        ],
        out_specs=pl.BlockSpec((1, 1, B), lambda i: (i, 0, 0)),
        out_shape=jax.ShapeDtypeStruct((nb, 1, B), jnp.float32),
    )(skeys, t, s0)[:, 0, :].T


def _cumsum_lanes(x):
    n = x.shape[-1]
    shift = 1
    while shift < n:
        z = jnp.zeros(x.shape[:-1] + (shift,), x.dtype)
        x = x + jnp.concatenate([z, x[..., :-shift]], axis=-1)
        shift *= 2
    return x


def _c2_body(skey_ref, sidx_ref, g_ref, t_ref, s0_ref, z_ref, carry_ref,
             k_ref, p_ref, maxv_ref, tok_ref):
    j = pl.program_id(0)
    col = j * _VT + jax.lax.broadcasted_iota(jnp.int32, skey_ref.shape, 1)
    valid = col < _V
    s = _key_to_logit(skey_ref[...]) / t_ref[...]
    e = jnp.exp(s - s0_ref[...])
    probs = jnp.where(valid, e / z_ref[...], 0.0)
    carry = carry_ref[0, 0, :][:, None]
    cum_excl = carry + _cumsum_lanes(probs) - probs
    mask = (col < k_ref[...]) & (cum_excl < p_ref[...])
    mask = mask | (col == 0)
    mask = mask & valid
    val = jnp.where(mask, s, _NEG_INF) + g_ref[...]
    val = jnp.where(valid, val, _NEG_INF)
    m = jnp.max(val, axis=1, keepdims=True)
    pos = jnp.min(jnp.where(val == m, col, jnp.int32(2**30)), axis=1,
                  keepdims=True)
    lpos = col == pos
    tok = jnp.max(jnp.where(lpos, sidx_ref[...], -1), axis=1, keepdims=True)
    maxv_ref[0, 0, :] = m[:, 0]
    tok_ref[0, 0, :] = tok[:, 0]


def _c2(skeys, sidx, g, t, s0, z, carry, k, p):
    B = skeys.shape[0]
    nb = _VP // _VT
    return pl.pallas_call(
        _c2_body,
        grid=(nb,),
        in_specs=[
            pl.BlockSpec((B, _VT), lambda i: (0, i)),
            pl.BlockSpec((B, _VT), lambda i: (0, i)),
            pl.BlockSpec((B, _VT), lambda i: (0, i)),
            pl.BlockSpec((B, 1), lambda i: (0, 0)),
            pl.BlockSpec((B, 1), lambda i: (0, 0)),
            pl.BlockSpec((B, 1), lambda i: (0, 0)),
            pl.BlockSpec((1, 1, B), lambda i: (i, 0, 0)),
            pl.BlockSpec((B, 1), lambda i: (0, 0)),
            pl.BlockSpec((B, 1), lambda i: (0, 0)),
        ],
        out_specs=[
            pl.BlockSpec((1, 1, B), lambda i: (i, 0, 0)),
            pl.BlockSpec((1, 1, B), lambda i: (i, 0, 0)),
        ],
        out_shape=[
            jax.ShapeDtypeStruct((nb, 1, B), jnp.float32),
            jax.ShapeDtypeStruct((nb, 1, B), jnp.int32),
        ],
    )(skeys, sidx, g, t, s0, z, carry, k, p)


def kernel(hidden_states, temperatures, top_ps, embedding, last_token_indices, top_ks):
    B = temperatures.shape[0]
    h = jnp.take(hidden_states, last_token_indices, axis=0)
    keys = _proj_keys(h, embedding)

    skeys, sidx = _sc_sort(keys)

    t = jnp.where(temperatures < _SAMPLING_EPS, 1.0, temperatures)[:, None]
    kk = jnp.maximum(top_ks, 1)[:, None]
    p = jnp.clip(top_ps, _SAMPLING_EPS, 1.0)[:, None]
    s0 = _key_to_logit(skeys[:, :1]) / t

    esum = _c1(skeys, t, s0)
    z = jnp.sum(esum, axis=1, keepdims=True)
    carry = (jnp.cumsum(esum, axis=1) - esum) / z
    carry = carry.T[:, None, :]

    g = jax.random.gumbel(jax.random.key(42), (B, _V), jnp.float32)
    maxv, tok = _c2(skeys, sidx, g, t, s0, z, carry, kk, p)
    maxv, tok = maxv[:, 0, :].T, tok[:, 0, :].T

    best = jnp.argmax(maxv, axis=1)
    tokens = jnp.take_along_axis(tok, best[:, None], axis=1)[:, 0]
    return tokens
